# Initial kernel scaffold; baseline (speedup 1.0000x reference)
import jax
import jax.numpy as jnp
from jax.experimental import pallas as pl


def kernel(hidden_states, context_mask, target_mask, mask_index, W, b, mask_token, pos_embed):
    raise NotImplementedError("write your pallas kernel here")


# speedup vs baseline: 3.0631x; 3.0631x over previous
import functools

import jax
import jax.numpy as jnp
from jax import lax
from jax.experimental import pallas as pl
from jax.experimental.pallas import tpu as pltpu
from jax.experimental.pallas import tpu_sc as plsc


def _sc_gather(idx_flat, table, n_rows, d, chunk, chunks_per_worker, nc, ns):
    per_worker = chunks_per_worker * chunk
    mesh = plsc.VectorSubcoreMesh(core_axis_name="c", subcore_axis_name="s")

    @functools.partial(
        pl.kernel,
        mesh=mesh,
        out_type=jax.ShapeDtypeStruct((n_rows, d), jnp.float32),
        scratch_types=[
            pltpu.VMEM((per_worker,), jnp.int32),
            pltpu.VMEM((chunk, d), jnp.float32),
            pltpu.SemaphoreType.DMA,
        ],
    )
    def gather_k(idx_hbm, table_hbm, out_hbm, idx_v, rows_v, sem):
        wid = lax.axis_index("s") * nc + lax.axis_index("c")
        base = wid * per_worker
        pltpu.sync_copy(idx_hbm.at[pl.ds(base, per_worker)], idx_v)
        for j in range(chunks_per_worker):
            pltpu.async_copy(table_hbm.at[idx_v.at[pl.ds(j * chunk, chunk)]], rows_v, sem).wait()
            pltpu.sync_copy(rows_v, out_hbm.at[pl.ds(base + j * chunk, chunk)])

    return gather_k(idx_flat, table)


def _tc_combine(hidden_states, W, b2, mt2, pos_all, n_ctx_blocks, n_blocks, rb):
    B, Kc, E = hidden_states.shape
    D = W.shape[1]
    K_total = pos_all.shape[1]

    def body(hs_ref, w_ref, b_ref, mt_ref, pos_ref, out_ref):
        r = pl.program_id(1)

        @pl.when(r < n_ctx_blocks)
        def _():
            acc = jax.lax.dot_general(
                hs_ref[0], w_ref[...], (((1,), (0,)), ((), ())),
                preferred_element_type=jnp.float32,
            )
            out_ref[0] = acc + b_ref[...] + pos_ref[0]

        @pl.when(r >= n_ctx_blocks)
        def _():
            out_ref[0] = pos_ref[0] + mt_ref[...]

    return pl.pallas_call(
        body,
        grid=(B, n_blocks),
        in_specs=[
            pl.BlockSpec((1, rb, E), lambda i, r: (i, jnp.minimum(r, n_ctx_blocks - 1), 0)),
            pl.BlockSpec((E, D), lambda i, r: (0, 0)),
            pl.BlockSpec((1, D), lambda i, r: (0, 0)),
            pl.BlockSpec((1, D), lambda i, r: (0, 0)),
            pl.BlockSpec((1, rb, D), lambda i, r: (i, r, 0)),
        ],
        out_specs=pl.BlockSpec((1, rb, D), lambda i, r: (i, r, 0)),
        out_shape=jax.ShapeDtypeStruct((B, K_total, D), jnp.float32),
    )(hidden_states, W, b2, mt2, pos_all)


def kernel(hidden_states, context_mask, target_mask, mask_index, W, b, mask_token, pos_embed):
    B, Kc, E = hidden_states.shape
    Kt = target_mask.shape[1]
    D = W.shape[1]
    K_total = Kc + Kt

    masks = jnp.concatenate([context_mask, target_mask], axis=1)

    nc, ns = 2, 16
    nw = nc * ns
    chunk = 128
    n_rows = B * K_total
    chunks_per_worker = n_rows // (nw * chunk)
    idx_flat = masks.reshape(n_rows)

    pos_all = _sc_gather(idx_flat, pos_embed, n_rows, D, chunk, chunks_per_worker, nc, ns)
    pos_all = pos_all.reshape(B, K_total, D)

    rb = 384
    n_ctx_blocks = Kc // rb
    n_blocks = K_total // rb
    b2 = b.reshape(1, D)
    mt2 = mask_token.reshape(1, D)

    embeddings = _tc_combine(hidden_states, W, b2, mt2, pos_all, n_ctx_blocks, n_blocks, rb)
    return (embeddings, masks)


# speedup vs baseline: 3.0635x; 1.0002x over previous
import functools

import jax
import jax.numpy as jnp
from jax import lax
from jax.experimental import pallas as pl
from jax.experimental.pallas import tpu as pltpu
from jax.experimental.pallas import tpu_sc as plsc


def _sc_gather(idx_flat, table, n_rows, d, chunk, chunks_per_worker, nc, ns):
    per_worker = chunks_per_worker * chunk
    mesh = plsc.VectorSubcoreMesh(core_axis_name="c", subcore_axis_name="s")

    @functools.partial(
        pl.kernel,
        mesh=mesh,
        out_type=jax.ShapeDtypeStruct((n_rows, d), jnp.float32),
        scratch_types=[
            pltpu.VMEM((per_worker,), jnp.int32),
            pltpu.VMEM((chunk, d), jnp.float32),
            pltpu.SemaphoreType.DMA,
        ],
    )
    def gather_k(idx_hbm, table_hbm, out_hbm, idx_v, rows_v, sem):
        wid = lax.axis_index("s") * nc + lax.axis_index("c")
        base = wid * per_worker
        pltpu.sync_copy(idx_hbm.at[pl.ds(base, per_worker)], idx_v)
        for j in range(chunks_per_worker):
            pltpu.async_copy(table_hbm.at[idx_v.at[pl.ds(j * chunk, chunk)]], rows_v, sem).wait()
            pltpu.sync_copy(rows_v, out_hbm.at[pl.ds(base + j * chunk, chunk)])

    return gather_k(idx_flat, table)


def _tc_combine(hidden_states, W, b2, mt2, pos_all, n_ctx_blocks, n_blocks, rb):
    B, Kc, E = hidden_states.shape
    D = W.shape[1]
    K_total = pos_all.shape[1]

    def body(hs_ref, w_ref, b_ref, mt_ref, pos_ref, out_ref):
        r = pl.program_id(1)

        @pl.when(r < n_ctx_blocks)
        def _():
            acc = jax.lax.dot_general(
                hs_ref[0].astype(jnp.bfloat16), w_ref[...].astype(jnp.bfloat16),
                (((1,), (0,)), ((), ())),
                preferred_element_type=jnp.float32,
            )
            out_ref[0] = acc + b_ref[...] + pos_ref[0]

        @pl.when(r >= n_ctx_blocks)
        def _():
            out_ref[0] = pos_ref[0] + mt_ref[...]

    return pl.pallas_call(
        body,
        grid=(B, n_blocks),
        in_specs=[
            pl.BlockSpec((1, rb, E), lambda i, r: (i, jnp.minimum(r, n_ctx_blocks - 1), 0)),
            pl.BlockSpec((E, D), lambda i, r: (0, 0)),
            pl.BlockSpec((1, D), lambda i, r: (0, 0)),
            pl.BlockSpec((1, D), lambda i, r: (0, 0)),
            pl.BlockSpec((1, rb, D), lambda i, r: (i, r, 0)),
        ],
        out_specs=pl.BlockSpec((1, rb, D), lambda i, r: (i, r, 0)),
        out_shape=jax.ShapeDtypeStruct((B, K_total, D), jnp.float32),
    )(hidden_states, W, b2, mt2, pos_all)


def kernel(hidden_states, context_mask, target_mask, mask_index, W, b, mask_token, pos_embed):
    B, Kc, E = hidden_states.shape
    Kt = target_mask.shape[1]
    D = W.shape[1]
    K_total = Kc + Kt

    masks = jnp.concatenate([context_mask, target_mask], axis=1)

    nc, ns = 2, 16
    nw = nc * ns
    chunk = 128
    n_rows = B * K_total
    chunks_per_worker = n_rows // (nw * chunk)
    idx_flat = masks.reshape(n_rows)

    pos_all = _sc_gather(idx_flat, pos_embed, n_rows, D, chunk, chunks_per_worker, nc, ns)
    pos_all = pos_all.reshape(B, K_total, D)

    rb = 384
    n_ctx_blocks = Kc // rb
    n_blocks = K_total // rb
    b2 = b.reshape(1, D)
    mt2 = mask_token.reshape(1, D)

    embeddings = _tc_combine(hidden_states, W, b2, mt2, pos_all, n_ctx_blocks, n_blocks, rb)
    return (embeddings, masks)


# speedup vs baseline: 3.8957x; 1.2716x over previous
import functools

import jax
import jax.numpy as jnp
from jax import lax
from jax.experimental import pallas as pl
from jax.experimental.pallas import tpu as pltpu
from jax.experimental.pallas import tpu_sc as plsc


def _sc_gather(idx_flat, table, n_rows, d, chunk, chunks_per_worker, nc, ns):
    per_worker = chunks_per_worker * chunk
    mesh = plsc.VectorSubcoreMesh(core_axis_name="c", subcore_axis_name="s")

    @functools.partial(
        pl.kernel,
        mesh=mesh,
        out_type=jax.ShapeDtypeStruct((n_rows, d), jnp.float32),
        scratch_types=[
            pltpu.VMEM((per_worker,), jnp.int32),
            pltpu.VMEM((chunk, d), jnp.float32),
            pltpu.SemaphoreType.DMA,
        ],
    )
    def gather_k(idx_hbm, table_hbm, out_hbm, idx_v, rows_v, sem):
        wid = lax.axis_index("s") * nc + lax.axis_index("c")
        base = wid * per_worker
        pltpu.sync_copy(idx_hbm.at[pl.ds(base, per_worker)], idx_v)
        for j in range(chunks_per_worker):
            pltpu.async_copy(table_hbm.at[idx_v.at[pl.ds(j * chunk, chunk)]], rows_v, sem).wait()
            pltpu.sync_copy(rows_v, out_hbm.at[pl.ds(base + j * chunk, chunk)])

    return gather_k(idx_flat, table)


def _tc_combine(hidden_states, W, pos_all, n_ctx_blocks, rb):
    B, Kc, E = hidden_states.shape
    D = W.shape[1]
    K_total = pos_all.shape[1]

    def body(hs_ref, w_ref, pos_ref, out_ref):
        acc = jax.lax.dot_general(
            hs_ref[0].astype(jnp.bfloat16), w_ref[...].astype(jnp.bfloat16),
            (((1,), (0,)), ((), ())),
            preferred_element_type=jnp.float32,
        )
        out_ref[0] = acc + pos_ref[0]

    return pl.pallas_call(
        body,
        grid=(B, n_ctx_blocks),
        in_specs=[
            pl.BlockSpec((1, rb, E), lambda i, r: (i, r, 0)),
            pl.BlockSpec((E, D), lambda i, r: (0, 0)),
            pl.BlockSpec((1, rb, D), lambda i, r: (i, r, 0)),
        ],
        out_specs=pl.BlockSpec((1, rb, D), lambda i, r: (i, r, 0)),
        out_shape=jax.ShapeDtypeStruct((B, K_total, D), jnp.float32),
        input_output_aliases={2: 0},
    )(hidden_states, W, pos_all)


def kernel(hidden_states, context_mask, target_mask, mask_index, W, b, mask_token, pos_embed):
    B, Kc, E = hidden_states.shape
    Kt = target_mask.shape[1]
    D = W.shape[1]
    P = pos_embed.shape[0]
    K_total = Kc + Kt

    masks = jnp.concatenate([context_mask, target_mask], axis=1)

    table = jnp.concatenate(
        [pos_embed + b[None, :], pos_embed + mask_token[0]], axis=0)
    idx = jnp.concatenate([context_mask, target_mask + P], axis=1)

    nc, ns = 2, 16
    nw = nc * ns
    chunk = 128
    n_rows = B * K_total
    chunks_per_worker = n_rows // (nw * chunk)

    pos_all = _sc_gather(idx.reshape(n_rows), table, n_rows, D, chunk,
                         chunks_per_worker, nc, ns)
    pos_all = pos_all.reshape(B, K_total, D)

    rb = 1152
    n_ctx_blocks = Kc // rb
    embeddings = _tc_combine(hidden_states, W, pos_all, n_ctx_blocks, rb)
    return (embeddings, masks)


# speedup vs baseline: 4.1433x; 1.0635x over previous
import functools

import jax
import jax.numpy as jnp
from jax import lax
from jax.experimental import pallas as pl
from jax.experimental.pallas import tpu as pltpu
from jax.experimental.pallas import tpu_sc as plsc


def _sc_gather2(idx_flat, table_c, table_t, n_rows, kc, kt, d, chunk,
                chunks_per_worker, nc, ns):
    per_worker = chunks_per_worker * chunk
    regions_per_batch = (kc + kt) // per_worker
    ctx_regions = kc // per_worker
    mesh = plsc.VectorSubcoreMesh(core_axis_name="c", subcore_axis_name="s")

    @functools.partial(
        pl.kernel,
        mesh=mesh,
        out_type=jax.ShapeDtypeStruct((n_rows, d), jnp.float32),
        scratch_types=[
            pltpu.VMEM((per_worker,), jnp.int32),
            pltpu.VMEM((chunk, d), jnp.float32),
            pltpu.SemaphoreType.DMA,
        ],
    )
    def gather_k(idx_hbm, tc_hbm, tt_hbm, out_hbm, idx_v, rows_v, sem):
        wid = lax.axis_index("s") * nc + lax.axis_index("c")
        base = wid * per_worker
        pltpu.sync_copy(idx_hbm.at[pl.ds(base, per_worker)], idx_v)
        is_ctx = lax.rem(wid, regions_per_batch) < ctx_regions

        @pl.when(is_ctx)
        def _():
            for j in range(chunks_per_worker):
                pltpu.async_copy(
                    tc_hbm.at[idx_v.at[pl.ds(j * chunk, chunk)]], rows_v, sem
                ).wait()
                pltpu.sync_copy(rows_v, out_hbm.at[pl.ds(base + j * chunk, chunk)])

        @pl.when(jnp.logical_not(is_ctx))
        def _():
            for j in range(chunks_per_worker):
                pltpu.async_copy(
                    tt_hbm.at[idx_v.at[pl.ds(j * chunk, chunk)]], rows_v, sem
                ).wait()
                pltpu.sync_copy(rows_v, out_hbm.at[pl.ds(base + j * chunk, chunk)])

    return gather_k(idx_flat, table_c, table_t)


def _tc_combine(hidden_states, W, pos_all, n_ctx_blocks, rb):
    B, Kc, E = hidden_states.shape
    D = W.shape[1]
    K_total = pos_all.shape[1]

    def body(hs_ref, w_ref, pos_ref, out_ref):
        acc = jax.lax.dot_general(
            hs_ref[0].astype(jnp.bfloat16), w_ref[...].astype(jnp.bfloat16),
            (((1,), (0,)), ((), ())),
            preferred_element_type=jnp.float32,
        )
        out_ref[0] = acc + pos_ref[0]

    return pl.pallas_call(
        body,
        grid=(B, n_ctx_blocks),
        in_specs=[
            pl.BlockSpec((1, rb, E), lambda i, r: (i, r, 0)),
            pl.BlockSpec((E, D), lambda i, r: (0, 0)),
            pl.BlockSpec((1, rb, D), lambda i, r: (i, r, 0)),
        ],
        out_specs=pl.BlockSpec((1, rb, D), lambda i, r: (i, r, 0)),
        out_shape=jax.ShapeDtypeStruct((B, K_total, D), jnp.float32),
        input_output_aliases={2: 0},
    )(hidden_states, W, pos_all)


def kernel(hidden_states, context_mask, target_mask, mask_index, W, b, mask_token, pos_embed):
    B, Kc, E = hidden_states.shape
    Kt = target_mask.shape[1]
    D = W.shape[1]
    K_total = Kc + Kt

    masks = jnp.concatenate([context_mask, target_mask], axis=1)

    table_c = pos_embed + b[None, :]
    table_t = pos_embed + mask_token[0]

    nc, ns = 2, 16
    nw = nc * ns
    chunk = 128
    n_rows = B * K_total
    n_ctx_rows = B * Kc
    chunks_per_worker = n_rows // (nw * chunk)

    pos_all = _sc_gather2(masks.reshape(n_rows), table_c, table_t, n_rows,
                          Kc, Kt, D, chunk, chunks_per_worker, nc, ns)
    pos_all = pos_all.reshape(B, K_total, D)

    rb = 1728
    n_ctx_blocks = Kc // rb
    embeddings = _tc_combine(hidden_states, W, pos_all, n_ctx_blocks, rb)
    return (embeddings, masks)
